# baseline (speedup 1.0000x reference)
import functools

import jax
import jax.numpy as jnp
import numpy as np
from jax.experimental import pallas as pl
from jax.experimental.pallas import tpu as pltpu
from jax.experimental.pallas import tpu_sc as plsc

END_BAND = 64
NUM_SC_CORES = 2
NUM_SUBCORES = 16
ROW_W = 256
WINDOW = 128


@functools.lru_cache(maxsize=None)
def _row_indices_host(num_bands, rows_per_band):
    try:
        with jax.ensure_compile_time_eval():
            perm_key = jax.random.key(42)
            perm = jax.random.permutation(perm_key, num_bands)[:END_BAND]
            bands = np.asarray(perm, dtype=np.int32)
        rows = (
            bands[:, None] * rows_per_band + np.arange(rows_per_band, dtype=np.int32)
        ).reshape(-1)
    except Exception:
        perm_key = jax.random.key(42)
        bands = jax.random.permutation(perm_key, num_bands)[:END_BAND].astype(
            jnp.int32
        )
        rows = (
            bands[:, None] * rows_per_band
            + jnp.arange(rows_per_band, dtype=jnp.int32)
        ).reshape(-1)
    return rows


@functools.lru_cache(maxsize=None)
def _make_sc_gather(n_rows_out, row_w):
    mesh = plsc.VectorSubcoreMesh(
        core_axis_name="core",
        subcore_axis_name="subcore",
        num_cores=NUM_SC_CORES,
        num_subcores=NUM_SUBCORES,
    )

    @functools.partial(
        pl.kernel,
        mesh=mesh,
        out_type=jax.ShapeDtypeStruct((n_rows_out, row_w), jnp.float32),
    )
    def sc_gather(x_hbm, ridx_hbm, out_hbm):
        def body(i_vmem, o_vmem):
            pltpu.sync_copy(x_hbm.at[i_vmem.at[0]], o_vmem)

        pltpu.emit_pipeline(
            body,
            grid=(n_rows_out // WINDOW,),
            in_specs=[
                pl.BlockSpec((1, WINDOW), index_map=lambda i: (0, i)),
            ],
            out_specs=[
                pl.BlockSpec((WINDOW, row_w), index_map=lambda i: (i, 0)),
            ],
            core_axis_name=("core", "subcore"),
            dimension_semantics=(pltpu.PARALLEL,),
        )(ridx_hbm, out_hbm)

    return sc_gather


def kernel(x):
    num_bands = x.shape[0]
    if num_bands <= END_BAND:
        return x
    H, W = x.shape[1], x.shape[2]
    rows_per_band = (H * W) // ROW_W
    rows = _row_indices_host(num_bands, rows_per_band)
    ridx = jnp.asarray(rows).reshape(1, -1)
    x2 = x.reshape(num_bands * rows_per_band, ROW_W)
    out2 = _make_sc_gather(END_BAND * rows_per_band, ROW_W)(x2, ridx)
    return out2.reshape(END_BAND, H, W)


# speedup vs baseline: 6.3422x; 6.3422x over previous
import functools

import jax
import jax.numpy as jnp
import numpy as np
from jax.experimental import pallas as pl
from jax.experimental.pallas import tpu as pltpu

END_BAND = 64
SPLIT = 2
RING = 24
LEAD = 12


def _band_indices(num_bands):
    try:
        with jax.ensure_compile_time_eval():
            perm_key = jax.random.key(42)
            perm = jax.random.permutation(perm_key, num_bands)[:END_BAND]
            return jnp.asarray(np.asarray(perm, dtype=np.int32))
    except Exception:
        perm_key = jax.random.key(42)
        perm = jax.random.permutation(perm_key, num_bands)[:END_BAND]
        return perm.astype(jnp.int32)


def _gather_kernel(idx_ref, x_hbm, o_hbm, buf, rsem, wsem):
    H = x_hbm.shape[1]
    hh = H // SPLIT
    total = END_BAND * SPLIT

    def read(t):
        b = t % RING
        return pltpu.make_async_copy(
            x_hbm.at[idx_ref[t // SPLIT], pl.ds((t % SPLIT) * hh, hh)],
            buf.at[b],
            rsem.at[b],
        )

    def write(t):
        b = t % RING
        return pltpu.make_async_copy(
            buf.at[b],
            o_hbm.at[t // SPLIT, pl.ds((t % SPLIT) * hh, hh)],
            wsem.at[b],
        )

    for t in range(min(LEAD, total)):
        read(t).start()
    for t in range(total):
        read(t).wait()
        write(t).start()
        tr = t + LEAD
        if tr < total:
            if tr >= RING:
                write(tr - RING).wait()
            read(tr).start()
    for t in range(max(0, total - RING), total):
        write(t).wait()


def kernel(x):
    num_bands = x.shape[0]
    if num_bands <= END_BAND:
        return x
    indices = _band_indices(num_bands)
    H, W = x.shape[1], x.shape[2]
    grid_spec = pltpu.PrefetchScalarGridSpec(
        num_scalar_prefetch=1,
        grid=(),
        in_specs=[pl.BlockSpec(memory_space=pl.ANY)],
        out_specs=pl.BlockSpec(memory_space=pl.ANY),
        scratch_shapes=[
            pltpu.VMEM((RING, H // SPLIT, W), jnp.float32),
            pltpu.SemaphoreType.DMA((RING,)),
            pltpu.SemaphoreType.DMA((RING,)),
        ],
    )
    return pl.pallas_call(
        _gather_kernel,
        grid_spec=grid_spec,
        out_shape=jax.ShapeDtypeStruct((END_BAND, H, W), x.dtype),
    )(indices, x)
